# baseline (speedup 1.0000x reference)
import jax
import jax.numpy as jnp
from jax.experimental import pallas as pl

B = 4096
D = 768
AD = 128
H = 512
NC = 5
E = 4
MH = 128
GH = 128

BLK = 512
PAD = 128
NEG = -1e30


def _dot(a, b):
    return jnp.dot(a, b, preferred_element_type=jnp.float32)


def _bdot(a, b16):
    return jnp.dot(a.astype(jnp.bfloat16), b16,
                   preferred_element_type=jnp.float32)


def _fused_body(cif_ref, label_ref, aud_ref, txt_ref, noise_ref,
                W_ap_ref, b_ap_ref, Wg1_ref, bg1_ref, Wg2_ref, bg2_ref,
                Wv_ref, bv_ref, Wo_ref, bo_ref, Wmg_ref, bmg_ref,
                eW1_ref, eb1_ref, eW2_ref, eb2_ref, ln_g_ref, ln_b_ref,
                W1_ref, b1_ref, W2_ref, b2_ref, Wfc_ref, bfc_ref,
                out_ref):
    cif = cif_ref[...]
    lane = jax.lax.broadcasted_iota(jnp.int32, (BLK, PAD), 1)

    txt = txt_ref[:, 0, :]
    x_aud = _dot(aud_ref[:, 0, :], W_ap_ref[...]) + b_ap_ref[...]

    labels = label_ref[...].reshape(BLK * 6, D)
    p_all = _bdot(labels, Wg1_ref[0:D, :]).reshape(BLK, 6, GH)
    g1 = jnp.zeros((BLK, GH), jnp.float32)
    for j in range(5):
        g1 = g1 + jnp.where(cif == j, 1.0, 0.0) * p_all[:, j, :]
    g1 = g1 + _bdot(x_aud, Wg1_ref[D:2 * D, :]) + _bdot(txt, Wg1_ref[2 * D:, :])
    g1 = jnp.maximum(g1 + bg1_ref[...], 0.0)
    glog = _dot(g1, Wg2_ref[...]) + bg2_ref[...]
    glm = jnp.where(lane < 3, glog, NEG)
    gmax = jnp.max(glm, axis=1, keepdims=True)
    ge = jnp.exp(glm - gmax)
    gw = ge / jnp.sum(ge, axis=1, keepdims=True)
    x_aud_s = gw[:, 1:2] * x_aud
    x_text = gw[:, 2:3] * txt

    y_att = _bdot(_bdot(x_aud_s, Wv_ref[...]) + bv_ref[...], Wo_ref[...]) + bo_ref[...]

    nm = _dot(y_att, Wmg_ref[...]) + bmg_ref[...] + noise_ref[...]
    nm = jnp.where(lane < E, nm, NEG)
    n0 = nm[:, 0:1]
    m1 = jnp.max(nm, axis=1, keepdims=True)
    am = jnp.min(jnp.where(nm == m1, lane, PAD), axis=1, keepdims=True)
    m2 = jnp.max(jnp.where(lane == am, NEG, nm), axis=1, keepdims=True)
    cnt = jnp.sum(jnp.where(nm > n0, 1.0, 0.0), axis=1, keepdims=True)
    s0 = jnp.where(cnt <= 1.5, jnp.exp(n0 - m1) / (1.0 + jnp.exp(m2 - m1)), 0.0)

    y16 = y_att.astype(jnp.bfloat16)
    acc = eb2_ref[...]
    for e in range(E):
        h_e = jnp.maximum(
            jnp.dot(y16, eW1_ref[e], preferred_element_type=jnp.float32)
            + eb1_ref[e:e + 1, :], 0.0)
        acc = acc + jnp.dot(h_e.astype(jnp.bfloat16), eW2_ref[e],
                            preferred_element_type=jnp.float32)
    y = s0 * acc

    ln_g = ln_g_ref[...]
    ln_b = ln_b_ref[...]
    ones_col = jnp.ones((D, 1), jnp.float32)
    inv_d = 1.0 / D

    def ln(v):
        mu = _dot(v, ones_col) * inv_d
        msq = _dot(v * v, ones_col) * inv_d
        var = msq - mu * mu
        return ln_g * (v - mu) * jax.lax.rsqrt(var + 1e-5) + ln_b

    x = x_text
    for _ in range(4):
        x = ln(y + x)
        y2 = _bdot(jnp.maximum(_bdot(x, W1_ref[...]) + b1_ref[...], 0.0),
                   W2_ref[...]) + b2_ref[...]
        x = ln(y2 + x)

    logits = _dot(x, Wfc_ref[...]) + bfc_ref[...]
    lm = jnp.where(lane < NC, logits, NEG)
    lmax = jnp.max(lm, axis=1, keepdims=True)
    le = jnp.exp(lm - lmax)
    out_ref[...] = le / jnp.sum(le, axis=1, keepdims=True)


def _padded(w, b, cols):
    wp = jnp.zeros((w.shape[0], PAD), jnp.float32).at[:, :cols].set(w)
    bp = jnp.zeros((1, PAD), jnp.float32).at[:, :cols].set(b)
    return wp, bp


@jax.jit
def kernel(cls_feats, label_feats, hiddens, audio_embedding, image_results,
           W_ap, b_ap, Wq, bq, Wk, bk, Wv, bv, Wo, bo, ln_g, ln_b,
           W1, b1, W2, b2, Wfc, bfc, eW1, eb1, eW2, eb2, Wmg, bmg,
           Wg1, bg1, Wg2, bg2):
    del cls_feats, Wq, bq, Wk, bk

    cif = image_results.astype(jnp.float32).reshape(B, 1)
    noise = jax.random.normal(jax.random.key(1), (B, E), jnp.float32) * 0.1
    noise_p = jnp.zeros((B, PAD), jnp.float32).at[:, :E].set(noise)

    Wg2p, bg2p = _padded(Wg2, bg2, 3)
    Wmgp, bmgp = _padded(Wmg, bmg, E)
    Wfcp, bfcp = _padded(Wfc, bfc, NC)
    bf = jnp.bfloat16
    eb2s = eb2.sum(0).reshape(1, D)
    Wg1h, Wvh, Woh, W1h, W2h, eW1h, eW2h = (
        w.astype(bf) for w in (Wg1, Wv, Wo, W1, W2, eW1, eW2))

    row2 = lambda v: v.reshape(1, -1)

    grid = (B // BLK,)
    bspec = lambda shape: pl.BlockSpec(shape, lambda i: (i, 0))
    wspec = lambda shape: pl.BlockSpec(shape, lambda i: (0, 0))
    wspec3 = lambda shape: pl.BlockSpec(shape, lambda i: (0, 0, 0))

    out = pl.pallas_call(
        _fused_body,
        grid=grid,
        in_specs=[
            bspec((BLK, 1)),
            pl.BlockSpec((BLK, 6, D), lambda i: (i, 0, 0)),
            pl.BlockSpec((BLK, 1, AD), lambda i: (i, 0, 0)),
            pl.BlockSpec((BLK, 1, D), lambda i: (i, 0, 0)),
            bspec((BLK, PAD)),
            wspec((AD, D)), wspec((1, D)),
            wspec((3 * D, GH)), wspec((1, GH)),
            wspec((GH, PAD)), wspec((1, PAD)),
            wspec((D, D)), wspec((1, D)),
            wspec((D, D)), wspec((1, D)),
            wspec((D, PAD)), wspec((1, PAD)),
            wspec3((E, D, MH)), wspec((E, MH)),
            wspec3((E, MH, D)), wspec((1, D)),
            wspec((1, D)), wspec((1, D)),
            wspec((D, H)), wspec((1, H)),
            wspec((H, D)), wspec((1, D)),
            wspec((D, PAD)), wspec((1, PAD)),
        ],
        out_specs=bspec((BLK, PAD)),
        out_shape=jax.ShapeDtypeStruct((B, PAD), jnp.float32),
    )(cif, label_feats, audio_embedding, hiddens, noise_p,
      W_ap, row2(b_ap), Wg1h, row2(bg1), Wg2p, bg2p,
      Wvh, row2(bv), Woh, row2(bo), Wmgp, bmgp,
      eW1h, eb1, eW2h, eb2s, row2(ln_g), row2(ln_b),
      W1h, row2(b1), W2h, row2(b2), Wfcp, bfcp)

    return out[:, :NC]


# speedup vs baseline: 1.4713x; 1.4713x over previous
import functools

import jax
import jax.numpy as jnp
from jax import lax
from jax.experimental import pallas as pl
from jax.experimental.pallas import tpu as pltpu
from jax.experimental.pallas import tpu_sc as plsc

B = 4096
D = 768
AD = 128
H = 512
NC = 5
E = 4
MH = 128
GH = 128

BLK = 512
PAD = 128
NEG = -1e30

def _sc_gather(ci, label_flat):
    info = plsc.get_sparse_core_info()
    ncs, nss, nl = info.num_cores, info.num_subcores, info.num_lanes
    nw = ncs * nss
    bpw = B // nw

    mesh = plsc.VectorSubcoreMesh(core_axis_name="c", subcore_axis_name="s")

    @functools.partial(
        pl.kernel, mesh=mesh,
        out_type=jax.ShapeDtypeStruct((B, D), jnp.float32),
        scratch_types=[
            pltpu.VMEM((bpw,), jnp.int32),
            pltpu.VMEM((bpw, D), jnp.float32),
            pltpu.SemaphoreType.DMA,
        ],
    )
    def gather_k(ci_hbm, label_hbm, out_hbm, idx_v, rows_v, sem):
        wid = lax.axis_index("s") * ncs + lax.axis_index("c")
        base = wid * bpw
        pltpu.sync_copy(ci_hbm.at[pl.ds(base, bpw)], idx_v)
        for i in range(bpw // nl):
            off = i * nl
            rid = lax.iota(jnp.int32, nl) + (base + off)
            idx_v[pl.ds(off, nl)] = idx_v[pl.ds(off, nl)] * B + rid
        pltpu.async_copy(label_hbm.at[idx_v], rows_v, sem).wait()
        pltpu.sync_copy(rows_v, out_hbm.at[pl.ds(base, bpw)])

    return gather_k(ci, label_flat)


def _dot(a, b):
    return jnp.dot(a, b, preferred_element_type=jnp.float32)


def _bdot(a, b16):
    return jnp.dot(a.astype(jnp.bfloat16), b16,
                   preferred_element_type=jnp.float32)


def _fused_body(cif_ref, gath_ref, aud_ref, txt_ref, noise_ref,
                W_ap_ref, b_ap_ref, Wg1_ref, bg1_ref, Wg2_ref, bg2_ref,
                Wv_ref, bv_ref, Wo_ref, bo_ref, Wmg_ref, bmg_ref,
                eW1_ref, eb1_ref, eW2_ref, eb2_ref, ln_g_ref, ln_b_ref,
                W1_ref, b1_ref, W2_ref, b2_ref, Wfc_ref, bfc_ref,
                out_ref):
    cif = cif_ref[...]
    lane = jax.lax.broadcasted_iota(jnp.int32, (BLK, PAD), 1)

    txt = txt_ref[:, 0, :]
    x_aud = _dot(aud_ref[:, 0, :], W_ap_ref[...]) + b_ap_ref[...]

    adjusted = jnp.where(cif == 5.0, 0.0, 1.0) * gath_ref[...]

    g1 = (_bdot(adjusted, Wg1_ref[0:D, :])
          + _bdot(x_aud, Wg1_ref[D:2 * D, :])
          + _bdot(txt, Wg1_ref[2 * D:, :]))
    g1 = jnp.maximum(g1 + bg1_ref[...], 0.0)
    glog = _dot(g1, Wg2_ref[...]) + bg2_ref[...]
    glm = jnp.where(lane < 3, glog, NEG)
    gmax = jnp.max(glm, axis=1, keepdims=True)
    ge = jnp.exp(glm - gmax)
    gw = ge / jnp.sum(ge, axis=1, keepdims=True)
    x_aud_s = gw[:, 1:2] * x_aud
    x_text = gw[:, 2:3] * txt

    y_att = _bdot(_bdot(x_aud_s, Wv_ref[...]) + bv_ref[...], Wo_ref[...]) + bo_ref[...]

    nm = _dot(y_att, Wmg_ref[...]) + bmg_ref[...] + noise_ref[...]
    nm = jnp.where(lane < E, nm, NEG)
    n0 = nm[:, 0:1]
    m1 = jnp.max(nm, axis=1, keepdims=True)
    am = jnp.min(jnp.where(nm == m1, lane, PAD), axis=1, keepdims=True)
    m2 = jnp.max(jnp.where(lane == am, NEG, nm), axis=1, keepdims=True)
    cnt = jnp.sum(jnp.where(nm > n0, 1.0, 0.0), axis=1, keepdims=True)
    s0 = jnp.where(cnt <= 1.5, jnp.exp(n0 - m1) / (1.0 + jnp.exp(m2 - m1)), 0.0)

    y16 = y_att.astype(jnp.bfloat16)
    acc = eb2_ref[...]
    for e in range(E):
        h_e = jnp.maximum(
            jnp.dot(y16, eW1_ref[e], preferred_element_type=jnp.float32)
            + eb1_ref[e:e + 1, :], 0.0)
        acc = acc + jnp.dot(h_e.astype(jnp.bfloat16), eW2_ref[e],
                            preferred_element_type=jnp.float32)
    y = s0 * acc

    ln_g = ln_g_ref[...]
    ln_b = ln_b_ref[...]
    ones_col = jnp.ones((D, 1), jnp.float32)
    inv_d = 1.0 / D

    def ln(v):
        mu = _dot(v, ones_col) * inv_d
        msq = _dot(v * v, ones_col) * inv_d
        var = msq - mu * mu
        return ln_g * (v - mu) * jax.lax.rsqrt(var + 1e-5) + ln_b

    x = x_text
    for _ in range(4):
        x = ln(y + x)
        y2 = _bdot(jnp.maximum(_bdot(x, W1_ref[...]) + b1_ref[...], 0.0),
                   W2_ref[...]) + b2_ref[...]
        x = ln(y2 + x)

    logits = _dot(x, Wfc_ref[...]) + bfc_ref[...]
    lm = jnp.where(lane < NC, logits, NEG)
    lmax = jnp.max(lm, axis=1, keepdims=True)
    le = jnp.exp(lm - lmax)
    out_ref[...] = le / jnp.sum(le, axis=1, keepdims=True)


def _padded(w, b, cols):
    wp = jnp.zeros((w.shape[0], PAD), jnp.float32).at[:, :cols].set(w)
    bp = jnp.zeros((1, PAD), jnp.float32).at[:, :cols].set(b)
    return wp, bp


@jax.jit
def kernel(cls_feats, label_feats, hiddens, audio_embedding, image_results,
           W_ap, b_ap, Wq, bq, Wk, bk, Wv, bv, Wo, bo, ln_g, ln_b,
           W1, b1, W2, b2, Wfc, bfc, eW1, eb1, eW2, eb2, Wmg, bmg,
           Wg1, bg1, Wg2, bg2):
    del cls_feats, Wq, bq, Wk, bk

    cif = image_results.astype(jnp.float32).reshape(B, 1)
    label_flat = label_feats.transpose(1, 0, 2).reshape(6 * B, D)
    gathered = _sc_gather(image_results, label_flat)
    noise_p = jnp.zeros((B, PAD), jnp.float32).at[:, :E].set(
        jax.random.normal(jax.random.key(1), (B, E), jnp.float32) * 0.1)

    Wg2p, bg2p = _padded(Wg2, bg2, 3)
    Wmgp, bmgp = _padded(Wmg, bmg, E)
    Wfcp, bfcp = _padded(Wfc, bfc, NC)
    bf = jnp.bfloat16
    eb2s = eb2.sum(0).reshape(1, D)
    Wg1h, Wvh, Woh, W1h, W2h, eW1h, eW2h = (
        w.astype(bf) for w in (Wg1, Wv, Wo, W1, W2, eW1, eW2))

    row2 = lambda v: v.reshape(1, -1)

    grid = (B // BLK,)
    bspec = lambda shape: pl.BlockSpec(shape, lambda i: (i, 0))
    wspec = lambda shape: pl.BlockSpec(shape, lambda i: (0, 0))
    wspec3 = lambda shape: pl.BlockSpec(shape, lambda i: (0, 0, 0))

    out = pl.pallas_call(
        _fused_body,
        grid=grid,
        in_specs=[
            bspec((BLK, 1)),
            bspec((BLK, D)),
            pl.BlockSpec((BLK, 1, AD), lambda i: (i, 0, 0)),
            pl.BlockSpec((BLK, 1, D), lambda i: (i, 0, 0)),
            bspec((BLK, PAD)),
            wspec((AD, D)), wspec((1, D)),
            wspec((3 * D, GH)), wspec((1, GH)),
            wspec((GH, PAD)), wspec((1, PAD)),
            wspec((D, D)), wspec((1, D)),
            wspec((D, D)), wspec((1, D)),
            wspec((D, PAD)), wspec((1, PAD)),
            wspec3((E, D, MH)), wspec((E, MH)),
            wspec3((E, MH, D)), wspec((1, D)),
            wspec((1, D)), wspec((1, D)),
            wspec((D, H)), wspec((1, H)),
            wspec((H, D)), wspec((1, D)),
            wspec((D, PAD)), wspec((1, PAD)),
        ],
        out_specs=bspec((BLK, PAD)),
        out_shape=jax.ShapeDtypeStruct((B, PAD), jnp.float32),
    )(cif, gathered, audio_embedding, hiddens, noise_p,
      W_ap, row2(b_ap), Wg1h, row2(bg1), Wg2p, bg2p,
      Wvh, row2(bv), Woh, row2(bo), Wmgp, bmgp,
      eW1h, eb1, eW2h, eb2s, row2(ln_g), row2(ln_b),
      W1h, row2(b1), W2h, row2(b2), Wfcp, bfcp)

    return out[:, :NC]


# speedup vs baseline: 1.5152x; 1.0298x over previous
import functools

import jax
import jax.numpy as jnp
from jax import lax
from jax.experimental import pallas as pl
from jax.experimental.pallas import tpu as pltpu
from jax.experimental.pallas import tpu_sc as plsc

B = 4096
D = 768
AD = 128
H = 512
NC = 5
E = 4
MH = 128
GH = 128

BLK = 1024
PAD = 128
NEG = -1e30

def _sc_gather(ci, label_flat):
    info = plsc.get_sparse_core_info()
    ncs, nss, nl = info.num_cores, info.num_subcores, info.num_lanes
    nw = ncs * nss
    bpw = B // nw

    mesh = plsc.VectorSubcoreMesh(core_axis_name="c", subcore_axis_name="s")

    @functools.partial(
        pl.kernel, mesh=mesh,
        out_type=jax.ShapeDtypeStruct((B, D), jnp.float32),
        scratch_types=[
            pltpu.VMEM((bpw,), jnp.int32),
            pltpu.VMEM((bpw, D), jnp.float32),
            pltpu.SemaphoreType.DMA,
        ],
    )
    def gather_k(ci_hbm, label_hbm, out_hbm, idx_v, rows_v, sem):
        wid = lax.axis_index("s") * ncs + lax.axis_index("c")
        base = wid * bpw
        pltpu.sync_copy(ci_hbm.at[pl.ds(base, bpw)], idx_v)
        for i in range(bpw // nl):
            off = i * nl
            rid = lax.iota(jnp.int32, nl) + (base + off)
            idx_v[pl.ds(off, nl)] = idx_v[pl.ds(off, nl)] * B + rid
        pltpu.async_copy(label_hbm.at[idx_v], rows_v, sem).wait()
        pltpu.sync_copy(rows_v, out_hbm.at[pl.ds(base, bpw)])

    return gather_k(ci, label_flat)


def _dot(a, b):
    return jnp.dot(a, b, preferred_element_type=jnp.float32)


def _bdot(a, b16):
    return jnp.dot(a.astype(jnp.bfloat16), b16,
                   preferred_element_type=jnp.float32)


def _fused_body(cif_ref, gath_ref, aud_ref, txt_ref, noise_ref,
                W_ap_ref, b_ap_ref, Wg1_ref, bg1_ref, Wg2_ref, bg2_ref,
                Wv_ref, bv_ref, Wo_ref, bo_ref, Wmg_ref, bmg_ref,
                eW1_ref, eb1_ref, eW2_ref, eb2_ref, ln_g_ref, ln_b_ref,
                W1_ref, b1_ref, W2_ref, b2_ref, Wfc_ref, bfc_ref,
                out_ref):
    cif = cif_ref[...]
    lane = jax.lax.broadcasted_iota(jnp.int32, (BLK, PAD), 1)

    txt = txt_ref[:, 0, :]
    x_aud = _dot(aud_ref[:, 0, :], W_ap_ref[...]) + b_ap_ref[...]

    adjusted = jnp.where(cif == 5.0, 0.0, 1.0) * gath_ref[...]

    g1 = (_bdot(adjusted, Wg1_ref[0:D, :])
          + _bdot(x_aud, Wg1_ref[D:2 * D, :])
          + _bdot(txt, Wg1_ref[2 * D:, :]))
    g1 = jnp.maximum(g1 + bg1_ref[...], 0.0)
    glog = _dot(g1, Wg2_ref[...]) + bg2_ref[...]
    glm = jnp.where(lane < 3, glog, NEG)
    gmax = jnp.max(glm, axis=1, keepdims=True)
    ge = jnp.exp(glm - gmax)
    gw = ge / jnp.sum(ge, axis=1, keepdims=True)
    x_aud_s = gw[:, 1:2] * x_aud
    x_text = gw[:, 2:3] * txt

    y_att = _bdot(_bdot(x_aud_s, Wv_ref[...]) + bv_ref[...], Wo_ref[...]) + bo_ref[...]

    nm = _dot(y_att, Wmg_ref[...]) + bmg_ref[...] + noise_ref[...]
    nm = jnp.where(lane < E, nm, NEG)
    n0 = nm[:, 0:1]
    m1 = jnp.max(nm, axis=1, keepdims=True)
    am = jnp.min(jnp.where(nm == m1, lane, PAD), axis=1, keepdims=True)
    m2 = jnp.max(jnp.where(lane == am, NEG, nm), axis=1, keepdims=True)
    cnt = jnp.sum(jnp.where(nm > n0, 1.0, 0.0), axis=1, keepdims=True)
    s0 = jnp.where(cnt <= 1.5, jnp.exp(n0 - m1) / (1.0 + jnp.exp(m2 - m1)), 0.0)

    y16 = y_att.astype(jnp.bfloat16)
    acc = eb2_ref[...]
    for e in range(E):
        h_e = jnp.maximum(
            jnp.dot(y16, eW1_ref[e], preferred_element_type=jnp.float32)
            + eb1_ref[e:e + 1, :], 0.0)
        acc = acc + jnp.dot(h_e.astype(jnp.bfloat16), eW2_ref[e],
                            preferred_element_type=jnp.float32)
    y = s0 * acc

    ln_g = ln_g_ref[...]
    ln_b = ln_b_ref[...]
    ones_col = jnp.ones((D, 1), jnp.bfloat16)
    inv_d = 1.0 / D

    def ln(v):
        v16 = v.astype(jnp.bfloat16)
        mu = jnp.dot(v16, ones_col, preferred_element_type=jnp.float32) * inv_d
        msq = jnp.dot(v16 * v16, ones_col,
                      preferred_element_type=jnp.float32) * inv_d
        var = msq - mu * mu
        return ln_g * (v - mu) * jax.lax.rsqrt(var + 1e-5) + ln_b

    x = x_text
    for _ in range(4):
        x = ln(y + x)
        y2 = _bdot(jnp.maximum(_bdot(x, W1_ref[...]) + b1_ref[...], 0.0),
                   W2_ref[...]) + b2_ref[...]
        x = ln(y2 + x)

    logits = _bdot(x, Wfc_ref[...]) + bfc_ref[...]
    lm = jnp.where(lane < NC, logits, NEG)
    lmax = jnp.max(lm, axis=1, keepdims=True)
    le = jnp.exp(lm - lmax)
    out_ref[...] = le / jnp.sum(le, axis=1, keepdims=True)


def _padded(w, b, cols):
    wp = jnp.zeros((w.shape[0], PAD), jnp.float32).at[:, :cols].set(w)
    bp = jnp.zeros((1, PAD), jnp.float32).at[:, :cols].set(b)
    return wp, bp


@jax.jit
def kernel(cls_feats, label_feats, hiddens, audio_embedding, image_results,
           W_ap, b_ap, Wq, bq, Wk, bk, Wv, bv, Wo, bo, ln_g, ln_b,
           W1, b1, W2, b2, Wfc, bfc, eW1, eb1, eW2, eb2, Wmg, bmg,
           Wg1, bg1, Wg2, bg2):
    del cls_feats, Wq, bq, Wk, bk

    cif = image_results.astype(jnp.float32).reshape(B, 1)
    label_flat = label_feats.transpose(1, 0, 2).reshape(6 * B, D)
    gathered = _sc_gather(image_results, label_flat)
    noise = jax.random.normal(jax.random.key(1), (B * E // PAD, PAD),
                              jnp.float32) * 0.1
    noise_p = jnp.zeros((B, PAD), jnp.float32).at[:, :E].set(
        noise.reshape(B, E))

    Wg2p, bg2p = _padded(Wg2, bg2, 3)
    Wmgp, bmgp = _padded(Wmg, bmg, E)
    Wfcp, bfcp = _padded(Wfc, bfc, NC)
    Wfcp = Wfcp.astype(jnp.bfloat16)
    bf = jnp.bfloat16
    eb2s = eb2.sum(0).reshape(1, D)
    Wg1h, Wvh, Woh, W1h, W2h, eW1h, eW2h = (
        w.astype(bf) for w in (Wg1, Wv, Wo, W1, W2, eW1, eW2))

    row2 = lambda v: v.reshape(1, -1)

    grid = (B // BLK,)
    bspec = lambda shape: pl.BlockSpec(shape, lambda i: (i, 0))
    wspec = lambda shape: pl.BlockSpec(shape, lambda i: (0, 0))
    wspec3 = lambda shape: pl.BlockSpec(shape, lambda i: (0, 0, 0))

    out = pl.pallas_call(
        _fused_body,
        grid=grid,
        in_specs=[
            bspec((BLK, 1)),
            bspec((BLK, D)),
            pl.BlockSpec((BLK, 1, AD), lambda i: (i, 0, 0)),
            pl.BlockSpec((BLK, 1, D), lambda i: (i, 0, 0)),
            bspec((BLK, PAD)),
            wspec((AD, D)), wspec((1, D)),
            wspec((3 * D, GH)), wspec((1, GH)),
            wspec((GH, PAD)), wspec((1, PAD)),
            wspec((D, D)), wspec((1, D)),
            wspec((D, D)), wspec((1, D)),
            wspec((D, PAD)), wspec((1, PAD)),
            wspec3((E, D, MH)), wspec((E, MH)),
            wspec3((E, MH, D)), wspec((1, D)),
            wspec((1, D)), wspec((1, D)),
            wspec((D, H)), wspec((1, H)),
            wspec((H, D)), wspec((1, D)),
            wspec((D, PAD)), wspec((1, PAD)),
        ],
        out_specs=bspec((BLK, PAD)),
        out_shape=jax.ShapeDtypeStruct((B, PAD), jnp.float32),
    )(cif, gathered, audio_embedding, hiddens, noise_p,
      W_ap, row2(b_ap), Wg1h, row2(bg1), Wg2p, bg2p,
      Wvh, row2(bv), Woh, row2(bo), Wmgp, bmgp,
      eW1h, eb1, eW2h, eb2s, row2(ln_g), row2(ln_b),
      W1h, row2(b1), W2h, row2(b2), Wfcp, bfcp)

    return out[:, :NC]
